# baseline (speedup 1.0000x reference)
import functools

import jax
import jax.numpy as jnp
from jax import lax
from jax.experimental import pallas as pl
from jax.experimental.pallas import tpu as pltpu
from jax.experimental.pallas import tpu_sc as plsc

B, N, D = 32, 512, 768
NUM_FREQ, NUM_TIME = 8, 64
ROWS = B * N
NC, NS = 2, 16
NW = NC * NS
RPT = ROWS // NW
CH = 16
NCH = RPT // CH
NBUF = 2
NGRP = NCH // NBUF


def _table_idx_body(fe_ref, te_ref, wa_ref, wb_ref, bias_ref, fp_ref, tp_ref,
                    table_ref, idx_ref):
    dn = (((1,), (1,)), ((), ()))
    wf = lax.dot_general(fe_ref[...], wa_ref[...], dn,
                         preferred_element_type=jnp.float32)
    wt = lax.dot_general(te_ref[...], wb_ref[...], dn,
                         preferred_element_type=jnp.float32)
    base = wt + bias_ref[...]
    for f in range(NUM_FREQ):
        table_ref[f * NUM_TIME:(f + 1) * NUM_TIME, :] = base + wf[f:f + 1, :]
    idx_ref[...] = fp_ref[...] * NUM_TIME + tp_ref[...]


@jax.jit
def _build_table_idx(fe, te, wa, wb, bias2d, fp, tp):
    return pl.pallas_call(
        _table_idx_body,
        out_shape=[
            jax.ShapeDtypeStruct((NUM_FREQ * NUM_TIME, D), jnp.float32),
            jax.ShapeDtypeStruct((B, N), jnp.int32),
        ],
    )(fe, te, wa, wb, bias2d, fp, tp)


def _sc_body(x_hbm, idx_hbm, table_hbm, out_hbm, idx_v, *bufs):
    xb = list(bufs[0:NBUF])
    gb = list(bufs[NBUF:2 * NBUF])
    ob = list(bufs[2 * NBUF:3 * NBUF])
    xs = list(bufs[3 * NBUF:4 * NBUF])
    gs = list(bufs[4 * NBUF:5 * NBUF])
    osm = list(bufs[5 * NBUF:6 * NBUF])

    wid = lax.axis_index("s") * NC + lax.axis_index("c")
    base = wid * RPT
    pltpu.sync_copy(idx_hbm.at[pl.ds(base, RPT)], idx_v)

    def issue(c, s):
        pltpu.async_copy(x_hbm.at[pl.ds(base + c * CH, CH)], xb[s], xs[s])
        pltpu.async_copy(
            table_hbm.at[idx_v.at[pl.ds(c * CH, CH)]], gb[s], gs[s])

    for s in range(NBUF):
        issue(s, s)

    def group(g, carry):
        for k in range(NBUF):
            s = k
            c = g * NBUF + k
            pltpu.make_async_copy(
                x_hbm.at[pl.ds(base, CH)], xb[s], xs[s]).wait()
            pltpu.make_async_copy(
                table_hbm.at[idx_v.at[pl.ds(0, CH)]], gb[s], gs[s]).wait()

            @pl.when(g > 0)
            def _():
                pltpu.make_async_copy(
                    ob[s], out_hbm.at[pl.ds(base, CH)], osm[s]).wait()

            def row(r, rc):
                for j in range(D // 16):
                    sl = pl.ds(j * 16, 16)
                    ob[s][r, sl] = xb[s][r, sl] + gb[s][r, sl]
                return rc

            lax.fori_loop(0, CH, row, 0)
            pltpu.async_copy(ob[s], out_hbm.at[pl.ds(base + c * CH, CH)],
                             osm[s])

            @pl.when(g < NGRP - 1)
            def _():
                issue(c + NBUF, s)
        return carry

    lax.fori_loop(0, NGRP, group, 0)
    for s in range(NBUF):
        pltpu.make_async_copy(
            ob[s], out_hbm.at[pl.ds(base, CH)], osm[s]).wait()


@jax.jit
def _sc_gather_add(xf, idx_flat, table):
    run = pl.kernel(
        _sc_body,
        out_type=jax.ShapeDtypeStruct((ROWS, D), jnp.float32),
        mesh=plsc.VectorSubcoreMesh(core_axis_name="c", subcore_axis_name="s"),
        scratch_types=[pltpu.VMEM((RPT,), jnp.int32)]
        + [pltpu.VMEM((CH, D), jnp.float32)] * (3 * NBUF)
        + [pltpu.SemaphoreType.DMA] * (3 * NBUF),
    )
    return run(xf, idx_flat, table)


def kernel(x, freq_pos, time_pos, freq_embedding, time_embedding, W, bias):
    wa = W[:, :D]
    wb = W[:, D:]
    table, idx = _build_table_idx(freq_embedding, time_embedding, wa, wb,
                                  bias.reshape(1, D),
                                  freq_pos.astype(jnp.int32),
                                  time_pos.astype(jnp.int32))
    out = _sc_gather_add(x.reshape(ROWS, D), idx.reshape(ROWS), table)
    return out.reshape(B, N, D)
